# Initial kernel scaffold; baseline (speedup 1.0000x reference)
import jax
import jax.numpy as jnp
from jax.experimental import pallas as pl


def kernel(x, batch, W1, b1, bn_gamma, bn_beta, W2, b2, L1w, L1b, L2w, L2b):
    raise NotImplementedError("write your pallas kernel here")


# speedup vs baseline: 4.6147x; 4.6147x over previous
import functools

import jax
import jax.numpy as jnp
from jax import lax
from jax.experimental import pallas as pl
from jax.experimental.pallas import tpu as pltpu
from jax.experimental.pallas import tpu_sc as plsc

N = 4096
D = 256
NB = 16
K = 8
BLK = 256
NBLK = N // BLK
F32 = jnp.float32
_INF = float("inf")

_NC, _NS = 2, 16
_NW = _NC * _NS
_EPW = (N * K) // _NW
_CH = 128
_NCHUNK = _EPW // _CH


def _prep_body(x_ref, wd_ref, wb_ref, xn_ref, a_ref, b_ref, sq_ref):
    x = x_ref[...]
    nrm = jnp.sqrt(jnp.sum(x * x, axis=1, keepdims=True))
    xn = x / jnp.maximum(nrm, 1e-12)
    xn_ref[...] = xn
    sq_ref[...] = jnp.sum(xn * xn, axis=1, keepdims=True)
    dn = (((1,), (1,)), ((), ()))
    a_ref[...] = lax.dot_general(xn, wd_ref[...], dn,
                                 preferred_element_type=F32,
                                 precision=lax.Precision.HIGHEST)
    b_ref[...] = lax.dot_general(xn, wb_ref[...], dn,
                                 preferred_element_type=F32,
                                 precision=lax.Precision.HIGHEST)


def _knn_body(xb_ref, br_ref, xf_ref, bc_ref, sqi_ref, sqj_ref, idx_ref):
    xb = xb_ref[...]
    xf = xf_ref[...]
    dn = (((1,), (1,)), ((), ()))
    dot = lax.dot_general(xb, xf, dn, preferred_element_type=F32)
    d2 = sqi_ref[...] + sqj_ref[...] - 2.0 * dot
    cross = br_ref[...] != bc_ref[...]
    d2 = jnp.where(cross, _INF, d2)
    jcol = lax.broadcasted_iota(jnp.int32, (BLK, N), 1)
    taken = jnp.zeros((BLK, N), jnp.bool_)
    cols = []
    for _ in range(K):
        cur = jnp.where(taken, _INF, d2)
        m = jnp.min(cur, axis=1, keepdims=True)
        hit = (cur == m) & jnp.logical_not(taken)
        am = jnp.min(jnp.where(hit, jcol, jnp.int32(N)), axis=1, keepdims=True)
        cols.append(am)
        taken = taken | (jcol == am)
    idx_ref[...] = jnp.concatenate(cols, axis=1)


def _gather_rows(table, idx_flat):
    mesh = plsc.VectorSubcoreMesh(core_axis_name="c", subcore_axis_name="s")

    @functools.partial(
        pl.kernel, mesh=mesh,
        out_type=jax.ShapeDtypeStruct((N * K, D), F32),
        scratch_types=[
            pltpu.VMEM((_CH,), jnp.int32),
            pltpu.VMEM((_CH, D), F32),
            pltpu.SemaphoreType.DMA,
        ],
    )
    def gk(table_hbm, idx_hbm, out_hbm, idx_v, rows_v, sem):
        wid = lax.axis_index("s") * _NC + lax.axis_index("c")
        base = wid * _EPW
        for c in range(_NCHUNK):
            off = base + c * _CH
            pltpu.sync_copy(idx_hbm.at[pl.ds(off, _CH)], idx_v)
            pltpu.async_copy(table_hbm.at[idx_v], rows_v, sem).wait()
            pltpu.sync_copy(rows_v, out_hbm.at[pl.ds(off, _CH)])

    return gk(table, idx_flat)


def _edge_h(bg_ref, a_ref, b1_ref):
    a3 = a_ref[...].reshape(BLK, 1, D)
    h = bg_ref[...] + a3 + b1_ref[...]
    return jnp.maximum(h, 0.0)


def _stats_body(bg_ref, a_ref, b1_ref, sum_ref, ssq_ref):
    i = pl.program_id(0)
    h = _edge_h(bg_ref, a_ref, b1_ref)
    hs = functools.reduce(jnp.add, [h[:, k, :] for k in range(K)])
    hq = functools.reduce(jnp.add, [h[:, k, :] * h[:, k, :] for k in range(K)])
    s = jnp.sum(hs, axis=0, keepdims=True)
    q = jnp.sum(hq, axis=0, keepdims=True)

    @pl.when(i == 0)
    def _():
        sum_ref[...] = jnp.zeros_like(sum_ref)
        ssq_ref[...] = jnp.zeros_like(ssq_ref)

    sum_ref[...] += s
    ssq_ref[...] += q


def _final_body(bg_ref, a_ref, b1_ref, s_ref, q_ref, gam_ref, bet_ref,
                w2_ref, b2_ref, bat_ref, l1w_ref, l1b_ref, l2w_ref, l2b_ref,
                out_ref, pool_ref):
    i = pl.program_id(0)
    dn = (((1,), (1,)), ((), ()))
    cnt = jnp.float32(N * K)
    mean = s_ref[...] / cnt
    var = q_ref[...] / cnt - mean * mean
    ascale = gam_ref[...] / jnp.sqrt(var + 1e-5)
    cshift = bet_ref[...] - ascale * mean
    h = _edge_h(bg_ref, a_ref, b1_ref)
    hflat = h.reshape(BLK * K, D) * ascale
    mm = lax.dot_general(hflat, w2_ref[...], dn, preferred_element_type=F32,
                         precision=lax.Precision.HIGHEST)
    dvec = lax.dot_general(cshift, w2_ref[...], dn,
                           preferred_element_type=F32,
                           precision=lax.Precision.HIGHEST) + b2_ref[...]
    h2 = (mm + dvec).reshape(BLK, K, D)
    node = functools.reduce(jnp.maximum, [h2[:, k, :] for k in range(K)])
    bat = bat_ref[...]
    rows = []
    for c in range(NB):
        w = jnp.where(bat == c, node, -_INF)
        rows.append(jnp.max(w, axis=0, keepdims=True))
    pool_blk = jnp.concatenate(rows, axis=0)

    @pl.when(i == 0)
    def _():
        pool_ref[...] = jnp.full_like(pool_ref, -_INF)

    pool_ref[...] = jnp.maximum(pool_ref[...], pool_blk)

    @pl.when(i == NBLK - 1)
    def _():
        p = pool_ref[...]
        t = lax.dot_general(p, l1w_ref[...], dn, preferred_element_type=F32,
                            precision=lax.Precision.HIGHEST) + l1b_ref[...]
        t = jnp.maximum(t, 0.0)
        o = lax.dot_general(t, l2w_ref[...], dn, preferred_element_type=F32,
                            precision=lax.Precision.HIGHEST) + l2b_ref[...]
        nrm = jnp.sqrt(jnp.sum(o * o, axis=1, keepdims=True))
        out_ref[...] = o / jnp.maximum(nrm, 1e-12)


def kernel(x, batch, W1, b1, bn_gamma, bn_beta, W2, b2, L1w, L1b, L2w, L2b):
    batch = batch.astype(jnp.int32)
    Wd = W1[:, :D] - W1[:, D:]
    Wb = W1[:, D:]

    xn, A, Bm, sq = pl.pallas_call(
        _prep_body,
        out_shape=[jax.ShapeDtypeStruct((N, D), F32)] * 3
        + [jax.ShapeDtypeStruct((N, 1), F32)],
    )(x, Wd, Wb)

    idx = pl.pallas_call(
        _knn_body,
        grid=(NBLK,),
        in_specs=[
            pl.BlockSpec((BLK, D), lambda i: (i, 0)),
            pl.BlockSpec((BLK, 1), lambda i: (i, 0)),
            pl.BlockSpec((N, D), lambda i: (0, 0)),
            pl.BlockSpec((1, N), lambda i: (0, 0)),
            pl.BlockSpec((BLK, 1), lambda i: (i, 0)),
            pl.BlockSpec((1, N), lambda i: (0, 0)),
        ],
        out_specs=pl.BlockSpec((BLK, K), lambda i: (i, 0)),
        out_shape=jax.ShapeDtypeStruct((N, K), jnp.int32),
    )(xn, batch.reshape(N, 1), xn, batch.reshape(1, N), sq, sq.reshape(1, N))

    Bg = _gather_rows(Bm, idx.reshape(N * K))
    Bg3 = Bg.reshape(N, K, D)
    b13 = b1.reshape(1, 1, D)

    ssum, ssq = pl.pallas_call(
        _stats_body,
        grid=(NBLK,),
        in_specs=[
            pl.BlockSpec((BLK, K, D), lambda i: (i, 0, 0)),
            pl.BlockSpec((BLK, D), lambda i: (i, 0)),
            pl.BlockSpec((1, 1, D), lambda i: (0, 0, 0)),
        ],
        out_specs=[pl.BlockSpec((1, D), lambda i: (0, 0))] * 2,
        out_shape=[jax.ShapeDtypeStruct((1, D), F32)] * 2,
    )(Bg3, A, b13)

    out = pl.pallas_call(
        _final_body,
        grid=(NBLK,),
        in_specs=[
            pl.BlockSpec((BLK, K, D), lambda i: (i, 0, 0)),
            pl.BlockSpec((BLK, D), lambda i: (i, 0)),
            pl.BlockSpec((1, 1, D), lambda i: (0, 0, 0)),
            pl.BlockSpec((1, D), lambda i: (0, 0)),
            pl.BlockSpec((1, D), lambda i: (0, 0)),
            pl.BlockSpec((1, D), lambda i: (0, 0)),
            pl.BlockSpec((1, D), lambda i: (0, 0)),
            pl.BlockSpec((D, D), lambda i: (0, 0)),
            pl.BlockSpec((1, D), lambda i: (0, 0)),
            pl.BlockSpec((BLK, 1), lambda i: (i, 0)),
            pl.BlockSpec((D, D), lambda i: (0, 0)),
            pl.BlockSpec((1, D), lambda i: (0, 0)),
            pl.BlockSpec((D, D), lambda i: (0, 0)),
            pl.BlockSpec((1, D), lambda i: (0, 0)),
        ],
        out_specs=pl.BlockSpec((NB, D), lambda i: (0, 0)),
        out_shape=jax.ShapeDtypeStruct((NB, D), F32),
        scratch_shapes=[pltpu.VMEM((NB, D), F32)],
    )(Bg3, A, b13, ssum, ssq, bn_gamma.reshape(1, D), bn_beta.reshape(1, D),
      W2, b2.reshape(1, D), batch.reshape(N, 1),
      L1w, L1b.reshape(1, D), L2w, L2b.reshape(1, D))

    return out


# speedup vs baseline: 5.8209x; 1.2614x over previous
import functools

import jax
import jax.numpy as jnp
from jax import lax
from jax.experimental import pallas as pl
from jax.experimental.pallas import tpu as pltpu
from jax.experimental.pallas import tpu_sc as plsc

N = 4096
D = 256
NB = 16
K = 8
BLK = 256
NBLK = N // BLK
F32 = jnp.float32
_INF = float("inf")

_NC, _NS = 2, 16
_NW = _NC * _NS
_EPW = (N * K) // _NW
_CH = 128
_NCHUNK = _EPW // _CH


def _prep_body(x_ref, wd_ref, wb_ref, xn_ref, a_ref, b_ref, sq_ref):
    x = x_ref[...]
    nrm = jnp.sqrt(jnp.sum(x * x, axis=1, keepdims=True))
    xn = x / jnp.maximum(nrm, 1e-12)
    xn_ref[...] = xn
    sq_ref[...] = jnp.sum(xn * xn, axis=1, keepdims=True)
    dn = (((1,), (1,)), ((), ()))
    a_ref[...] = lax.dot_general(xn, wd_ref[...], dn,
                                 preferred_element_type=F32,
                                 precision=lax.Precision.HIGHEST)
    b_ref[...] = lax.dot_general(xn, wb_ref[...], dn,
                                 preferred_element_type=F32,
                                 precision=lax.Precision.HIGHEST)


def _knn_body(xb_ref, br_ref, xf_ref, bc_ref, sqi_ref, sqj_ref, idx_ref):
    xb = xb_ref[...]
    xf = xf_ref[...]
    dn = (((1,), (1,)), ((), ()))
    dot = lax.dot_general(xb, xf, dn, preferred_element_type=F32)
    d2 = sqi_ref[...] + sqj_ref[...] - 2.0 * dot
    cross = br_ref[...] != bc_ref[...]
    work = jnp.where(cross, -1e30, -d2)
    jcol = lax.broadcasted_iota(jnp.int32, (BLK, N), 1)
    cols = []
    for _ in range(K):
        m = jnp.max(work, axis=1, keepdims=True)
        am = jnp.min(jnp.where(work == m, jcol, jnp.int32(N)),
                     axis=1, keepdims=True)
        cols.append(am)
        work = jnp.where(jcol == am, -_INF, work)
    idx_ref[...] = jnp.concatenate(cols, axis=1)


def _gather_rows(table, idx_flat):
    mesh = plsc.VectorSubcoreMesh(core_axis_name="c", subcore_axis_name="s")

    @functools.partial(
        pl.kernel, mesh=mesh,
        out_type=jax.ShapeDtypeStruct((N * K, D), F32),
        scratch_types=[
            pltpu.VMEM((_CH,), jnp.int32),
            pltpu.VMEM((_CH, D), F32),
            pltpu.SemaphoreType.DMA,
        ],
    )
    def gk(table_hbm, idx_hbm, out_hbm, idx_v, rows_v, sem):
        wid = lax.axis_index("s") * _NC + lax.axis_index("c")
        base = wid * _EPW
        for c in range(_NCHUNK):
            off = base + c * _CH
            pltpu.sync_copy(idx_hbm.at[pl.ds(off, _CH)], idx_v)
            pltpu.async_copy(table_hbm.at[idx_v], rows_v, sem).wait()
            pltpu.sync_copy(rows_v, out_hbm.at[pl.ds(off, _CH)])

    return gk(table, idx_flat)


def _edge_h(bg_ref, a_ref, b1_ref):
    a3 = a_ref[...].reshape(BLK, 1, D)
    h = bg_ref[...] + a3 + b1_ref[...]
    return jnp.maximum(h, 0.0)


def _stats_body(bg_ref, a_ref, b1_ref, sum_ref, ssq_ref):
    i = pl.program_id(0)
    h = _edge_h(bg_ref, a_ref, b1_ref)
    hs = functools.reduce(jnp.add, [h[:, k, :] for k in range(K)])
    hq = functools.reduce(jnp.add, [h[:, k, :] * h[:, k, :] for k in range(K)])
    s = jnp.sum(hs, axis=0, keepdims=True)
    q = jnp.sum(hq, axis=0, keepdims=True)

    @pl.when(i == 0)
    def _():
        sum_ref[...] = jnp.zeros_like(sum_ref)
        ssq_ref[...] = jnp.zeros_like(ssq_ref)

    sum_ref[...] += s
    ssq_ref[...] += q


def _final_body(bg_ref, a_ref, b1_ref, s_ref, q_ref, gam_ref, bet_ref,
                w2_ref, b2_ref, bat_ref, l1w_ref, l1b_ref, l2w_ref, l2b_ref,
                out_ref, pool_ref):
    i = pl.program_id(0)
    dn = (((1,), (1,)), ((), ()))
    cnt = jnp.float32(N * K)
    mean = s_ref[...] / cnt
    var = q_ref[...] / cnt - mean * mean
    ascale = gam_ref[...] / jnp.sqrt(var + 1e-5)
    cshift = bet_ref[...] - ascale * mean
    h = _edge_h(bg_ref, a_ref, b1_ref)
    hflat = h.reshape(BLK * K, D) * ascale
    mm = lax.dot_general(hflat, w2_ref[...], dn, preferred_element_type=F32,
                         precision=lax.Precision.HIGHEST)
    dvec = lax.dot_general(cshift, w2_ref[...], dn,
                           preferred_element_type=F32,
                           precision=lax.Precision.HIGHEST) + b2_ref[...]
    h2 = (mm + dvec).reshape(BLK, K, D)
    node = functools.reduce(jnp.maximum, [h2[:, k, :] for k in range(K)])
    bat = bat_ref[...]
    rows = []
    for c in range(NB):
        w = jnp.where(bat == c, node, -_INF)
        rows.append(jnp.max(w, axis=0, keepdims=True))
    pool_blk = jnp.concatenate(rows, axis=0)

    @pl.when(i == 0)
    def _():
        pool_ref[...] = jnp.full_like(pool_ref, -_INF)

    pool_ref[...] = jnp.maximum(pool_ref[...], pool_blk)

    @pl.when(i == NBLK - 1)
    def _():
        p = pool_ref[...]
        t = lax.dot_general(p, l1w_ref[...], dn, preferred_element_type=F32,
                            precision=lax.Precision.HIGHEST) + l1b_ref[...]
        t = jnp.maximum(t, 0.0)
        o = lax.dot_general(t, l2w_ref[...], dn, preferred_element_type=F32,
                            precision=lax.Precision.HIGHEST) + l2b_ref[...]
        nrm = jnp.sqrt(jnp.sum(o * o, axis=1, keepdims=True))
        out_ref[...] = o / jnp.maximum(nrm, 1e-12)


def kernel(x, batch, W1, b1, bn_gamma, bn_beta, W2, b2, L1w, L1b, L2w, L2b):
    batch = batch.astype(jnp.int32)
    Wd = W1[:, :D] - W1[:, D:]
    Wb = W1[:, D:]

    xn, A, Bm, sq = pl.pallas_call(
        _prep_body,
        out_shape=[jax.ShapeDtypeStruct((N, D), F32)] * 3
        + [jax.ShapeDtypeStruct((N, 1), F32)],
    )(x, Wd, Wb)

    idx = pl.pallas_call(
        _knn_body,
        grid=(NBLK,),
        in_specs=[
            pl.BlockSpec((BLK, D), lambda i: (i, 0)),
            pl.BlockSpec((BLK, 1), lambda i: (i, 0)),
            pl.BlockSpec((N, D), lambda i: (0, 0)),
            pl.BlockSpec((1, N), lambda i: (0, 0)),
            pl.BlockSpec((BLK, 1), lambda i: (i, 0)),
            pl.BlockSpec((1, N), lambda i: (0, 0)),
        ],
        out_specs=pl.BlockSpec((BLK, K), lambda i: (i, 0)),
        out_shape=jax.ShapeDtypeStruct((N, K), jnp.int32),
    )(xn, batch.reshape(N, 1), xn, batch.reshape(1, N), sq, sq.reshape(1, N))

    Bg = _gather_rows(Bm, idx.reshape(N * K))
    Bg3 = Bg.reshape(N, K, D)
    b13 = b1.reshape(1, 1, D)

    ssum, ssq = pl.pallas_call(
        _stats_body,
        grid=(NBLK,),
        in_specs=[
            pl.BlockSpec((BLK, K, D), lambda i: (i, 0, 0)),
            pl.BlockSpec((BLK, D), lambda i: (i, 0)),
            pl.BlockSpec((1, 1, D), lambda i: (0, 0, 0)),
        ],
        out_specs=[pl.BlockSpec((1, D), lambda i: (0, 0))] * 2,
        out_shape=[jax.ShapeDtypeStruct((1, D), F32)] * 2,
    )(Bg3, A, b13)

    out = pl.pallas_call(
        _final_body,
        grid=(NBLK,),
        in_specs=[
            pl.BlockSpec((BLK, K, D), lambda i: (i, 0, 0)),
            pl.BlockSpec((BLK, D), lambda i: (i, 0)),
            pl.BlockSpec((1, 1, D), lambda i: (0, 0, 0)),
            pl.BlockSpec((1, D), lambda i: (0, 0)),
            pl.BlockSpec((1, D), lambda i: (0, 0)),
            pl.BlockSpec((1, D), lambda i: (0, 0)),
            pl.BlockSpec((1, D), lambda i: (0, 0)),
            pl.BlockSpec((D, D), lambda i: (0, 0)),
            pl.BlockSpec((1, D), lambda i: (0, 0)),
            pl.BlockSpec((BLK, 1), lambda i: (i, 0)),
            pl.BlockSpec((D, D), lambda i: (0, 0)),
            pl.BlockSpec((1, D), lambda i: (0, 0)),
            pl.BlockSpec((D, D), lambda i: (0, 0)),
            pl.BlockSpec((1, D), lambda i: (0, 0)),
        ],
        out_specs=pl.BlockSpec((NB, D), lambda i: (0, 0)),
        out_shape=jax.ShapeDtypeStruct((NB, D), F32),
        scratch_shapes=[pltpu.VMEM((NB, D), F32)],
    )(Bg3, A, b13, ssum, ssq, bn_gamma.reshape(1, D), bn_beta.reshape(1, D),
      W2, b2.reshape(1, D), batch.reshape(N, 1),
      L1w, L1b.reshape(1, D), L2w, L2b.reshape(1, D))

    return out


# speedup vs baseline: 7.3153x; 1.2567x over previous
import functools

import jax
import jax.numpy as jnp
from jax import lax
from jax.experimental import pallas as pl
from jax.experimental.pallas import tpu as pltpu
from jax.experimental.pallas import tpu_sc as plsc

N = 4096
D = 256
NB = 16
K = 8
BLK = 256
NBLK = N // BLK
F32 = jnp.float32
_INF = float("inf")

_NC, _NS = 2, 16
_NW = _NC * _NS
_EPW = (N * K) // _NW
_CH = 128
_NCHUNK = _EPW // _CH


def _mm3(xv, w):
    dn = (((1,), (1,)), ((), ()))
    xh32 = xv.astype(jnp.bfloat16).astype(F32)
    xh = xh32.astype(jnp.bfloat16)
    xl = (xv - xh32).astype(jnp.bfloat16)
    wh32 = w.astype(jnp.bfloat16).astype(F32)
    wh = wh32.astype(jnp.bfloat16)
    wl = (w - wh32).astype(jnp.bfloat16)
    t1 = lax.dot_general(xh, wh, dn, preferred_element_type=F32)
    t2 = lax.dot_general(xl, wh, dn, preferred_element_type=F32)
    t3 = lax.dot_general(xh, wl, dn, preferred_element_type=F32)
    return t1 + t2 + t3


def _prep_body(x_ref, wd_ref, wb_ref, xn_ref, a_ref, b_ref, sq_ref):
    x = x_ref[...]
    nrm = jnp.sqrt(jnp.sum(x * x, axis=1, keepdims=True))
    xn = x / jnp.maximum(nrm, 1e-12)
    xn_ref[...] = xn
    sq_ref[...] = jnp.sum(xn * xn, axis=1, keepdims=True)
    dn = (((1,), (1,)), ((), ()))
    a_ref[...] = _mm3(xn, wd_ref[...])
    b_ref[...] = _mm3(xn, wb_ref[...])


def _knn_body(xb_ref, br_ref, xf_ref, bc_ref, sqi_ref, sqj_ref, idx_ref):
    xb = xb_ref[...]
    xf = xf_ref[...]
    dn = (((1,), (1,)), ((), ()))
    dot = lax.dot_general(xb, xf, dn, preferred_element_type=F32)
    d2 = sqi_ref[...] + sqj_ref[...] - 2.0 * dot
    cross = br_ref[...] != bc_ref[...]
    work = jnp.where(cross, -1e30, -d2)
    jcol = lax.broadcasted_iota(jnp.int32, (BLK, N), 1)
    cols = []
    for _ in range(K):
        m = jnp.max(work, axis=1, keepdims=True)
        am = jnp.min(jnp.where(work == m, jcol, jnp.int32(N)),
                     axis=1, keepdims=True)
        cols.append(am)
        work = jnp.where(jcol == am, -_INF, work)
    idx_ref[...] = jnp.concatenate(cols, axis=1)


def _gather_rows(table, idx_flat):
    mesh = plsc.VectorSubcoreMesh(core_axis_name="c", subcore_axis_name="s")

    @functools.partial(
        pl.kernel, mesh=mesh,
        out_type=jax.ShapeDtypeStruct((N * K, D), F32),
        scratch_types=[
            pltpu.VMEM((_CH,), jnp.int32),
            pltpu.VMEM((_CH, D), F32),
            pltpu.SemaphoreType.DMA,
        ],
    )
    def gk(table_hbm, idx_hbm, out_hbm, idx_v, rows_v, sem):
        wid = lax.axis_index("s") * _NC + lax.axis_index("c")
        base = wid * _EPW
        for c in range(_NCHUNK):
            off = base + c * _CH
            pltpu.sync_copy(idx_hbm.at[pl.ds(off, _CH)], idx_v)
            pltpu.async_copy(table_hbm.at[idx_v], rows_v, sem).wait()
            pltpu.sync_copy(rows_v, out_hbm.at[pl.ds(off, _CH)])

    return gk(table, idx_flat)


def _edge_h(bg_ref, a_ref, b1_ref):
    h = bg_ref[...] + a_ref[...].reshape(1, BLK, D) + b1_ref[...]
    return jnp.maximum(h, 0.0)


def _stats_body(bg_ref, a_ref, b1_ref, sum_ref, ssq_ref):
    i = pl.program_id(0)
    h = _edge_h(bg_ref, a_ref, b1_ref)
    hs = functools.reduce(jnp.add, [h[k] for k in range(K)])
    hq = functools.reduce(jnp.add, [h[k] * h[k] for k in range(K)])
    s = jnp.sum(hs.reshape(BLK // 8, 8, D), axis=0)
    q = jnp.sum(hq.reshape(BLK // 8, 8, D), axis=0)

    @pl.when(i == 0)
    def _():
        sum_ref[...] = jnp.zeros_like(sum_ref)
        ssq_ref[...] = jnp.zeros_like(ssq_ref)

    sum_ref[...] += s
    ssq_ref[...] += q


def _final_body(bg_ref, a_ref, b1_ref, s_ref, q_ref, gam_ref, bet_ref,
                w2_ref, b2_ref, bat_ref, l1w_ref, l1b_ref, l2w_ref, l2b_ref,
                out_ref, pool_ref):
    i = pl.program_id(0)
    dn = (((1,), (1,)), ((), ()))
    cnt = jnp.float32(N * K)
    mean = jnp.sum(s_ref[...], axis=0, keepdims=True) / cnt
    var = jnp.sum(q_ref[...], axis=0, keepdims=True) / cnt - mean * mean
    ascale = gam_ref[...] / jnp.sqrt(var + 1e-5)
    cshift = bet_ref[...] - ascale * mean
    h = _edge_h(bg_ref, a_ref, b1_ref)
    hflat = h.reshape(BLK * K, D) * ascale
    mm = lax.dot_general(hflat, w2_ref[...], dn, preferred_element_type=F32,
                         precision=lax.Precision.HIGHEST)
    dvec = lax.dot_general(cshift, w2_ref[...], dn,
                           preferred_element_type=F32,
                           precision=lax.Precision.HIGHEST) + b2_ref[...]
    h2 = (mm + dvec).reshape(K, BLK, D)
    node = functools.reduce(jnp.maximum, [h2[k] for k in range(K)])
    bat = bat_ref[...]
    rows = []
    for c in range(NB):
        w = jnp.where(bat == c, node, -_INF)
        rows.append(jnp.max(w.reshape(BLK // 8, 8, D), axis=0))
    pool_blk = jnp.concatenate(rows, axis=0)

    @pl.when(i == 0)
    def _():
        pool_ref[...] = jnp.full_like(pool_ref, -_INF)

    pool_ref[...] = jnp.maximum(pool_ref[...], pool_blk)

    @pl.when(i == NBLK - 1)
    def _():
        p8 = pool_ref[...].reshape(NB, 8, D)
        p = functools.reduce(jnp.maximum, [p8[:, k, :] for k in range(8)])
        t = lax.dot_general(p, l1w_ref[...], dn, preferred_element_type=F32,
                            precision=lax.Precision.HIGHEST) + l1b_ref[...]
        t = jnp.maximum(t, 0.0)
        o = lax.dot_general(t, l2w_ref[...], dn, preferred_element_type=F32,
                            precision=lax.Precision.HIGHEST) + l2b_ref[...]
        nrm = jnp.sqrt(jnp.sum(o * o, axis=1, keepdims=True))
        out_ref[...] = o / jnp.maximum(nrm, 1e-12)


def kernel(x, batch, W1, b1, bn_gamma, bn_beta, W2, b2, L1w, L1b, L2w, L2b):
    batch = batch.astype(jnp.int32)
    Wd = W1[:, :D] - W1[:, D:]
    Wb = W1[:, D:]

    xn, A, Bm, sq = pl.pallas_call(
        _prep_body,
        out_shape=[jax.ShapeDtypeStruct((N, D), F32)] * 3
        + [jax.ShapeDtypeStruct((N, 1), F32)],
    )(x, Wd, Wb)

    idx = pl.pallas_call(
        _knn_body,
        grid=(NBLK,),
        in_specs=[
            pl.BlockSpec((BLK, D), lambda i: (i, 0)),
            pl.BlockSpec((BLK, 1), lambda i: (i, 0)),
            pl.BlockSpec((N, D), lambda i: (0, 0)),
            pl.BlockSpec((1, N), lambda i: (0, 0)),
            pl.BlockSpec((BLK, 1), lambda i: (i, 0)),
            pl.BlockSpec((1, N), lambda i: (0, 0)),
        ],
        out_specs=pl.BlockSpec((BLK, K), lambda i: (i, 0)),
        out_shape=jax.ShapeDtypeStruct((N, K), jnp.int32),
    )(xn, batch.reshape(N, 1), xn, batch.reshape(1, N), sq, sq.reshape(1, N))

    Bg = _gather_rows(Bm, idx.T.reshape(N * K))
    Bg3 = Bg.reshape(K, N, D)
    b13 = b1.reshape(1, 1, D)

    ssum, ssq = pl.pallas_call(
        _stats_body,
        grid=(NBLK,),
        in_specs=[
            pl.BlockSpec((K, BLK, D), lambda i: (0, i, 0)),
            pl.BlockSpec((BLK, D), lambda i: (i, 0)),
            pl.BlockSpec((1, 1, D), lambda i: (0, 0, 0)),
        ],
        out_specs=[pl.BlockSpec((8, D), lambda i: (0, 0))] * 2,
        out_shape=[jax.ShapeDtypeStruct((8, D), F32)] * 2,
    )(Bg3, A, b13)

    out = pl.pallas_call(
        _final_body,
        grid=(NBLK,),
        in_specs=[
            pl.BlockSpec((K, BLK, D), lambda i: (0, i, 0)),
            pl.BlockSpec((BLK, D), lambda i: (i, 0)),
            pl.BlockSpec((1, 1, D), lambda i: (0, 0, 0)),
            pl.BlockSpec((8, D), lambda i: (0, 0)),
            pl.BlockSpec((8, D), lambda i: (0, 0)),
            pl.BlockSpec((1, D), lambda i: (0, 0)),
            pl.BlockSpec((1, D), lambda i: (0, 0)),
            pl.BlockSpec((D, D), lambda i: (0, 0)),
            pl.BlockSpec((1, D), lambda i: (0, 0)),
            pl.BlockSpec((BLK, 1), lambda i: (i, 0)),
            pl.BlockSpec((D, D), lambda i: (0, 0)),
            pl.BlockSpec((1, D), lambda i: (0, 0)),
            pl.BlockSpec((D, D), lambda i: (0, 0)),
            pl.BlockSpec((1, D), lambda i: (0, 0)),
        ],
        out_specs=pl.BlockSpec((NB, D), lambda i: (0, 0)),
        out_shape=jax.ShapeDtypeStruct((NB, D), F32),
        scratch_shapes=[pltpu.VMEM((NB * 8, D), F32)],
    )(Bg3, A, b13, ssum, ssq, bn_gamma.reshape(1, D), bn_beta.reshape(1, D),
      W2, b2.reshape(1, D), batch.reshape(N, 1),
      L1w, L1b.reshape(1, D), L2w, L2b.reshape(1, D))

    return out


# speedup vs baseline: 8.9867x; 1.2285x over previous
import functools

import jax
import jax.numpy as jnp
from jax import lax
from jax.experimental import pallas as pl
from jax.experimental.pallas import tpu as pltpu
from jax.experimental.pallas import tpu_sc as plsc

N = 4096
D = 256
NB = 16
K = 8
BLK = 256
NBLK = N // BLK
F32 = jnp.float32
_INF = float("inf")

_NC, _NS = 2, 16
_NW = _NC * _NS
_EPW = (N * K) // _NW
_CH = 128
_NCHUNK = _EPW // _CH


def _mm3(xv, w):
    dn = (((1,), (1,)), ((), ()))
    xh32 = xv.astype(jnp.bfloat16).astype(F32)
    xh = xh32.astype(jnp.bfloat16)
    xl = (xv - xh32).astype(jnp.bfloat16)
    wh32 = w.astype(jnp.bfloat16).astype(F32)
    wh = wh32.astype(jnp.bfloat16)
    wl = (w - wh32).astype(jnp.bfloat16)
    t1 = lax.dot_general(xh, wh, dn, preferred_element_type=F32)
    t2 = lax.dot_general(xl, wh, dn, preferred_element_type=F32)
    t3 = lax.dot_general(xh, wl, dn, preferred_element_type=F32)
    return t1 + t2 + t3


def _prep_body(x_ref, wd_ref, wb_ref, xn_ref, a_ref, b_ref, sq_ref):
    x = x_ref[...]
    nrm = jnp.sqrt(jnp.sum(x * x, axis=1, keepdims=True))
    xn = x / jnp.maximum(nrm, 1e-12)
    xn_ref[...] = xn
    sq_ref[...] = jnp.sum(xn * xn, axis=1, keepdims=True)
    dn = (((1,), (1,)), ((), ()))
    a_ref[...] = _mm3(xn, wd_ref[...])
    b_ref[...] = _mm3(xn, wb_ref[...])


def _knn_body(xb_ref, br_ref, xf_ref, bc_ref, sqi_ref, sqj_ref, idx_ref):
    xb = xb_ref[...]
    xf = xf_ref[...]
    dn = (((1,), (1,)), ((), ()))
    dot = lax.dot_general(xb, xf, dn, preferred_element_type=F32)
    d2 = sqi_ref[...] + sqj_ref[...] - 2.0 * dot
    cross = br_ref[...] != bc_ref[...]
    work = jnp.where(cross, -1e30, -d2)
    jcol = lax.broadcasted_iota(jnp.int32, (BLK, N), 1)
    cols = []
    for _ in range(K):
        m = jnp.max(work, axis=1, keepdims=True)
        am = jnp.min(jnp.where(work == m, jcol, jnp.int32(N)),
                     axis=1, keepdims=True)
        cols.append(am)
        work = jnp.where(jcol == am, -_INF, work)
    idx_ref[...] = jnp.concatenate(cols, axis=1)


WIN = 1536
_WSUB = 512


def _knn_win_body(u_ref, xb_ref, br_ref, sqi_ref, x0, x1, x2,
                  c0, c1, c2, q0, q1, q2, idx_ref):
    i = pl.program_id(0)
    base = u_ref[i] * _WSUB
    xfw = jnp.concatenate([x0[...], x1[...], x2[...]], axis=0)
    bcw = jnp.concatenate([c0[...], c1[...], c2[...]], axis=1)
    sqw = jnp.concatenate([q0[...], q1[...], q2[...]], axis=1)
    dn = (((1,), (1,)), ((), ()))
    dot = lax.dot_general(xb_ref[...], xfw, dn, preferred_element_type=F32)
    d2 = sqi_ref[...] + sqw - 2.0 * dot
    work = jnp.where(br_ref[...] != bcw, -1e30, -d2)
    jcol = lax.broadcasted_iota(jnp.int32, (BLK, WIN), 1) + base
    cols = []
    for _ in range(K):
        m = jnp.max(work, axis=1, keepdims=True)
        am = jnp.min(jnp.where(work == m, jcol, jnp.int32(N)),
                     axis=1, keepdims=True)
        cols.append(am)
        work = jnp.where(jcol == am, -_INF, work)
    idx_ref[...] = jnp.concatenate(cols, axis=1)


def _knn_windowed(u, xn, batch_r, batch_c, sqi, sqj):
    grid_spec = pltpu.PrefetchScalarGridSpec(
        num_scalar_prefetch=1,
        grid=(NBLK,),
        in_specs=[
            pl.BlockSpec((BLK, D), lambda i, u: (i, 0)),
            pl.BlockSpec((BLK, 1), lambda i, u: (i, 0)),
            pl.BlockSpec((BLK, 1), lambda i, u: (i, 0)),
            pl.BlockSpec((_WSUB, D), lambda i, u: (u[i], 0)),
            pl.BlockSpec((_WSUB, D), lambda i, u: (u[i] + 1, 0)),
            pl.BlockSpec((_WSUB, D), lambda i, u: (u[i] + 2, 0)),
            pl.BlockSpec((1, _WSUB), lambda i, u: (0, u[i])),
            pl.BlockSpec((1, _WSUB), lambda i, u: (0, u[i] + 1)),
            pl.BlockSpec((1, _WSUB), lambda i, u: (0, u[i] + 2)),
            pl.BlockSpec((1, _WSUB), lambda i, u: (0, u[i])),
            pl.BlockSpec((1, _WSUB), lambda i, u: (0, u[i] + 1)),
            pl.BlockSpec((1, _WSUB), lambda i, u: (0, u[i] + 2)),
        ],
        out_specs=pl.BlockSpec((BLK, K), lambda i, u: (i, 0)),
    )
    return pl.pallas_call(
        _knn_win_body, grid_spec=grid_spec,
        out_shape=jax.ShapeDtypeStruct((N, K), jnp.int32),
    )(u, xn, batch_r, sqi, xn, xn, xn,
      batch_c, batch_c, batch_c, sqj, sqj, sqj)


def _gather_rows(table, idx_flat):
    mesh = plsc.VectorSubcoreMesh(core_axis_name="c", subcore_axis_name="s")

    @functools.partial(
        pl.kernel, mesh=mesh,
        out_type=jax.ShapeDtypeStruct((N * K, D), F32),
        scratch_types=[
            pltpu.VMEM((_CH,), jnp.int32),
            pltpu.VMEM((_CH, D), F32),
            pltpu.SemaphoreType.DMA,
        ],
    )
    def gk(table_hbm, idx_hbm, out_hbm, idx_v, rows_v, sem):
        wid = lax.axis_index("s") * _NC + lax.axis_index("c")
        base = wid * _EPW
        for c in range(_NCHUNK):
            off = base + c * _CH
            pltpu.sync_copy(idx_hbm.at[pl.ds(off, _CH)], idx_v)
            pltpu.async_copy(table_hbm.at[idx_v], rows_v, sem).wait()
            pltpu.sync_copy(rows_v, out_hbm.at[pl.ds(off, _CH)])

    return gk(table, idx_flat)


def _edge_h(bg_ref, a_ref, b1_ref):
    h = bg_ref[...] + a_ref[...].reshape(1, BLK, D) + b1_ref[...]
    return jnp.maximum(h, 0.0)


def _stats_body(bg_ref, a_ref, b1_ref, sum_ref, ssq_ref):
    i = pl.program_id(0)
    h = _edge_h(bg_ref, a_ref, b1_ref)
    hs = functools.reduce(jnp.add, [h[k] for k in range(K)])
    hq = functools.reduce(jnp.add, [h[k] * h[k] for k in range(K)])
    s = jnp.sum(hs.reshape(BLK // 8, 8, D), axis=0)
    q = jnp.sum(hq.reshape(BLK // 8, 8, D), axis=0)

    @pl.when(i == 0)
    def _():
        sum_ref[...] = jnp.zeros_like(sum_ref)
        ssq_ref[...] = jnp.zeros_like(ssq_ref)

    sum_ref[...] += s
    ssq_ref[...] += q


def _final_body(bg_ref, a_ref, b1_ref, s_ref, q_ref, gam_ref, bet_ref,
                w2_ref, b2_ref, bat_ref, l1w_ref, l1b_ref, l2w_ref, l2b_ref,
                out_ref, pool_ref):
    i = pl.program_id(0)
    dn = (((1,), (1,)), ((), ()))
    cnt = jnp.float32(N * K)
    mean = jnp.sum(s_ref[...], axis=0, keepdims=True) / cnt
    var = jnp.sum(q_ref[...], axis=0, keepdims=True) / cnt - mean * mean
    ascale = gam_ref[...] / jnp.sqrt(var + 1e-5)
    cshift = bet_ref[...] - ascale * mean
    h = _edge_h(bg_ref, a_ref, b1_ref)
    hflat = h.reshape(BLK * K, D) * ascale
    mm = lax.dot_general(hflat, w2_ref[...], dn, preferred_element_type=F32,
                         precision=lax.Precision.HIGHEST)
    dvec = lax.dot_general(cshift, w2_ref[...], dn,
                           preferred_element_type=F32,
                           precision=lax.Precision.HIGHEST) + b2_ref[...]
    h2 = (mm + dvec).reshape(K, BLK, D)
    node = functools.reduce(jnp.maximum, [h2[k] for k in range(K)])
    bat = bat_ref[...]
    rows = []
    for c in range(NB):
        w = jnp.where(bat == c, node, -_INF)
        rows.append(jnp.max(w.reshape(BLK // 8, 8, D), axis=0))
    pool_blk = jnp.concatenate(rows, axis=0)

    @pl.when(i == 0)
    def _():
        pool_ref[...] = jnp.full_like(pool_ref, -_INF)

    pool_ref[...] = jnp.maximum(pool_ref[...], pool_blk)

    @pl.when(i == NBLK - 1)
    def _():
        p8 = pool_ref[...].reshape(NB, 8, D)
        p = functools.reduce(jnp.maximum, [p8[:, k, :] for k in range(8)])
        t = lax.dot_general(p, l1w_ref[...], dn, preferred_element_type=F32,
                            precision=lax.Precision.HIGHEST) + l1b_ref[...]
        t = jnp.maximum(t, 0.0)
        o = lax.dot_general(t, l2w_ref[...], dn, preferred_element_type=F32,
                            precision=lax.Precision.HIGHEST) + l2b_ref[...]
        nrm = jnp.sqrt(jnp.sum(o * o, axis=1, keepdims=True))
        out_ref[...] = o / jnp.maximum(nrm, 1e-12)


def kernel(x, batch, W1, b1, bn_gamma, bn_beta, W2, b2, L1w, L1b, L2w, L2b):
    batch = batch.astype(jnp.int32)
    Wd = W1[:, :D] - W1[:, D:]
    Wb = W1[:, D:]

    xn, A, Bm, sq = pl.pallas_call(
        _prep_body,
        out_shape=[jax.ShapeDtypeStruct((N, D), F32)] * 3
        + [jax.ShapeDtypeStruct((N, 1), F32)],
    )(x, Wd, Wb)

    rng = jnp.arange(NB, dtype=batch.dtype)
    cs = jnp.searchsorted(batch, rng, side='left').astype(jnp.int32)
    ce = jnp.searchsorted(batch, rng, side='right').astype(jnp.int32)
    bfirst = batch[::BLK]
    blast = batch[BLK - 1::BLK]
    starts = cs[bfirst]
    ends = ce[blast]
    u = jnp.minimum(starts // _WSUB, (N - WIN) // _WSUB).astype(jnp.int32)
    fits = jnp.all(ends - u * _WSUB <= WIN) & jnp.all(ce - cs >= K)

    batch_r = batch.reshape(N, 1)
    batch_c = batch.reshape(1, N)
    sqj = sq.reshape(1, N)

    def _full_knn():
        return pl.pallas_call(
            _knn_body,
            grid=(NBLK,),
            in_specs=[
                pl.BlockSpec((BLK, D), lambda i: (i, 0)),
                pl.BlockSpec((BLK, 1), lambda i: (i, 0)),
                pl.BlockSpec((N, D), lambda i: (0, 0)),
                pl.BlockSpec((1, N), lambda i: (0, 0)),
                pl.BlockSpec((BLK, 1), lambda i: (i, 0)),
                pl.BlockSpec((1, N), lambda i: (0, 0)),
            ],
            out_specs=pl.BlockSpec((BLK, K), lambda i: (i, 0)),
            out_shape=jax.ShapeDtypeStruct((N, K), jnp.int32),
        )(xn, batch_r, xn, batch_c, sq, sqj)

    idx = lax.cond(fits,
                   lambda: _knn_windowed(u, xn, batch_r, batch_c, sq, sqj),
                   _full_knn)

    Bg = _gather_rows(Bm, idx.T.reshape(N * K))
    Bg3 = Bg.reshape(K, N, D)
    b13 = b1.reshape(1, 1, D)

    ssum, ssq = pl.pallas_call(
        _stats_body,
        grid=(NBLK,),
        in_specs=[
            pl.BlockSpec((K, BLK, D), lambda i: (0, i, 0)),
            pl.BlockSpec((BLK, D), lambda i: (i, 0)),
            pl.BlockSpec((1, 1, D), lambda i: (0, 0, 0)),
        ],
        out_specs=[pl.BlockSpec((8, D), lambda i: (0, 0))] * 2,
        out_shape=[jax.ShapeDtypeStruct((8, D), F32)] * 2,
    )(Bg3, A, b13)

    out = pl.pallas_call(
        _final_body,
        grid=(NBLK,),
        in_specs=[
            pl.BlockSpec((K, BLK, D), lambda i: (0, i, 0)),
            pl.BlockSpec((BLK, D), lambda i: (i, 0)),
            pl.BlockSpec((1, 1, D), lambda i: (0, 0, 0)),
            pl.BlockSpec((8, D), lambda i: (0, 0)),
            pl.BlockSpec((8, D), lambda i: (0, 0)),
            pl.BlockSpec((1, D), lambda i: (0, 0)),
            pl.BlockSpec((1, D), lambda i: (0, 0)),
            pl.BlockSpec((D, D), lambda i: (0, 0)),
            pl.BlockSpec((1, D), lambda i: (0, 0)),
            pl.BlockSpec((BLK, 1), lambda i: (i, 0)),
            pl.BlockSpec((D, D), lambda i: (0, 0)),
            pl.BlockSpec((1, D), lambda i: (0, 0)),
            pl.BlockSpec((D, D), lambda i: (0, 0)),
            pl.BlockSpec((1, D), lambda i: (0, 0)),
        ],
        out_specs=pl.BlockSpec((NB, D), lambda i: (0, 0)),
        out_shape=jax.ShapeDtypeStruct((NB, D), F32),
        scratch_shapes=[pltpu.VMEM((NB * 8, D), F32)],
    )(Bg3, A, b13, ssum, ssq, bn_gamma.reshape(1, D), bn_beta.reshape(1, D),
      W2, b2.reshape(1, D), batch.reshape(N, 1),
      L1w, L1b.reshape(1, D), L2w, L2b.reshape(1, D))

    return out


# speedup vs baseline: 9.2044x; 1.0242x over previous
import functools

import jax
import jax.numpy as jnp
from jax import lax
from jax.experimental import pallas as pl
from jax.experimental.pallas import tpu as pltpu
from jax.experimental.pallas import tpu_sc as plsc

N = 4096
D = 256
NB = 16
K = 8
BLK = 256
NBLK = N // BLK
F32 = jnp.float32
_INF = float("inf")

_NC, _NS = 2, 16
_NW = _NC * _NS
_EPW = (N * K) // _NW
_CH = 128
_NCHUNK = _EPW // _CH


def _mm3(xv, w):
    dn = (((1,), (1,)), ((), ()))
    xh32 = xv.astype(jnp.bfloat16).astype(F32)
    xh = xh32.astype(jnp.bfloat16)
    xl = (xv - xh32).astype(jnp.bfloat16)
    wh32 = w.astype(jnp.bfloat16).astype(F32)
    wh = wh32.astype(jnp.bfloat16)
    wl = (w - wh32).astype(jnp.bfloat16)
    t1 = lax.dot_general(xh, wh, dn, preferred_element_type=F32)
    t2 = lax.dot_general(xl, wh, dn, preferred_element_type=F32)
    t3 = lax.dot_general(xh, wl, dn, preferred_element_type=F32)
    return t1 + t2 + t3


def _prep_body(x_ref, wd_ref, wb_ref, xn_ref, a_ref, b_ref, sq_ref):
    x = x_ref[...]
    nrm = jnp.sqrt(jnp.sum(x * x, axis=1, keepdims=True))
    xn = x / jnp.maximum(nrm, 1e-12)
    xn_ref[...] = xn
    sq_ref[...] = jnp.sum(xn * xn, axis=1, keepdims=True)
    dn = (((1,), (1,)), ((), ()))
    a_ref[...] = _mm3(xn, wd_ref[...])
    b_ref[...] = _mm3(xn, wb_ref[...])


def _knn_body(xb_ref, br_ref, xf_ref, bc_ref, sqi_ref, sqj_ref, idx_ref):
    xb = xb_ref[...]
    xf = xf_ref[...]
    dn = (((1,), (1,)), ((), ()))
    dot = lax.dot_general(xb, xf, dn, preferred_element_type=F32)
    d2 = sqi_ref[...] + sqj_ref[...] - 2.0 * dot
    cross = br_ref[...] != bc_ref[...]
    work = jnp.where(cross, -1e30, -d2)
    jcol = lax.broadcasted_iota(jnp.int32, (BLK, N), 1)
    cols = []
    for _ in range(K):
        m = jnp.max(work, axis=1, keepdims=True)
        am = jnp.min(jnp.where(work == m, jcol, jnp.int32(N)),
                     axis=1, keepdims=True)
        cols.append(am)
        work = jnp.where(jcol == am, -_INF, work)
    idx_ref[...] = jnp.concatenate(cols, axis=1)


WIN = 1536
_WSUB = 512


def _knn_win_body(u_ref, xb_ref, br_ref, sqi_ref, x0, x1, x2,
                  c0, c1, c2, q0, q1, q2, idx_ref):
    i = pl.program_id(0)
    base = u_ref[i] * _WSUB
    xfw = jnp.concatenate([x0[...], x1[...], x2[...]], axis=0)
    bcw = jnp.concatenate([c0[...], c1[...], c2[...]], axis=1)
    sqw = jnp.concatenate([q0[...], q1[...], q2[...]], axis=1)
    dn = (((1,), (1,)), ((), ()))
    dot = lax.dot_general(xb_ref[...], xfw, dn, preferred_element_type=F32)
    d2 = sqi_ref[...] + sqw - 2.0 * dot
    work = jnp.where(br_ref[...] != bcw, -1e30, -d2)
    jcol = lax.broadcasted_iota(jnp.int32, (BLK, WIN), 1) + base
    cols = []
    for _ in range(K):
        m = jnp.max(work, axis=1, keepdims=True)
        am = jnp.min(jnp.where(work == m, jcol, jnp.int32(N)),
                     axis=1, keepdims=True)
        cols.append(am)
        work = jnp.where(jcol == am, -_INF, work)
    idx_ref[...] = jnp.concatenate(cols, axis=1)


def _knn_windowed(u, xn, batch_r, batch_c, sqi, sqj):
    grid_spec = pltpu.PrefetchScalarGridSpec(
        num_scalar_prefetch=1,
        grid=(NBLK,),
        in_specs=[
            pl.BlockSpec((BLK, D), lambda i, u: (i, 0)),
            pl.BlockSpec((BLK, 1), lambda i, u: (i, 0)),
            pl.BlockSpec((BLK, 1), lambda i, u: (i, 0)),
            pl.BlockSpec((_WSUB, D), lambda i, u: (u[i], 0)),
            pl.BlockSpec((_WSUB, D), lambda i, u: (u[i] + 1, 0)),
            pl.BlockSpec((_WSUB, D), lambda i, u: (u[i] + 2, 0)),
            pl.BlockSpec((1, _WSUB), lambda i, u: (0, u[i])),
            pl.BlockSpec((1, _WSUB), lambda i, u: (0, u[i] + 1)),
            pl.BlockSpec((1, _WSUB), lambda i, u: (0, u[i] + 2)),
            pl.BlockSpec((1, _WSUB), lambda i, u: (0, u[i])),
            pl.BlockSpec((1, _WSUB), lambda i, u: (0, u[i] + 1)),
            pl.BlockSpec((1, _WSUB), lambda i, u: (0, u[i] + 2)),
        ],
        out_specs=pl.BlockSpec((BLK, K), lambda i, u: (i, 0)),
    )
    return pl.pallas_call(
        _knn_win_body, grid_spec=grid_spec,
        out_shape=jax.ShapeDtypeStruct((N, K), jnp.int32),
    )(u, xn, batch_r, sqi, xn, xn, xn,
      batch_c, batch_c, batch_c, sqj, sqj, sqj)


def _gather_rows(table, idx_flat):
    mesh = plsc.VectorSubcoreMesh(core_axis_name="c", subcore_axis_name="s")

    @functools.partial(
        pl.kernel, mesh=mesh,
        out_type=jax.ShapeDtypeStruct((N * K, D), F32),
        scratch_types=[
            pltpu.VMEM((_CH,), jnp.int32),
            pltpu.VMEM((_CH, D), F32),
            pltpu.SemaphoreType.DMA,
        ],
    )
    def gk(table_hbm, idx_hbm, out_hbm, idx_v, rows_v, sem):
        wid = lax.axis_index("s") * _NC + lax.axis_index("c")
        base = wid * _EPW
        for c in range(_NCHUNK):
            off = base + c * _CH
            pltpu.sync_copy(idx_hbm.at[pl.ds(off, _CH)], idx_v)
            pltpu.async_copy(table_hbm.at[idx_v], rows_v, sem).wait()
            pltpu.sync_copy(rows_v, out_hbm.at[pl.ds(off, _CH)])

    return gk(table, idx_flat)


def _edge_h(bg_ref, a_ref, b1_ref):
    h = bg_ref[...] + a_ref[...].reshape(1, BLK, D) + b1_ref[...]
    return jnp.maximum(h, 0.0)


def _edge_body(bg_ref, a_ref, b1_ref, gam_ref, bet_ref,
               w2_ref, b2_ref, bat_ref, l1w_ref, l1b_ref, l2w_ref, l2b_ref,
               out_ref, sum_ref, ssq_ref, pool_ref):
    i = pl.program_id(0)
    dn = (((1,), (1,)), ((), ()))
    h = _edge_h(bg_ref, a_ref, b1_ref)

    @pl.when(i == 0)
    def _():
        sum_ref[...] = jnp.zeros_like(sum_ref)
        ssq_ref[...] = jnp.zeros_like(ssq_ref)
        pool_ref[...] = jnp.full_like(pool_ref, -_INF)

    @pl.when(i < NBLK)
    def _():
        hs = functools.reduce(jnp.add, [h[k] for k in range(K)])
        hq = functools.reduce(jnp.add, [h[k] * h[k] for k in range(K)])
        sum_ref[...] += jnp.sum(hs.reshape(BLK // 8, 8, D), axis=0)
        ssq_ref[...] += jnp.sum(hq.reshape(BLK // 8, 8, D), axis=0)

    @pl.when(i >= NBLK)
    def _():
        cnt = jnp.float32(N * K)
        mean = jnp.sum(sum_ref[...], axis=0, keepdims=True) / cnt
        var = jnp.sum(ssq_ref[...], axis=0, keepdims=True) / cnt - mean * mean
        ascale = gam_ref[...] / jnp.sqrt(var + 1e-5)
        cshift = bet_ref[...] - ascale * mean
        hflat = h.reshape(BLK * K, D) * ascale
        mm = _mm3(hflat, w2_ref[...])
        dvec = lax.dot_general(cshift, w2_ref[...], dn,
                               preferred_element_type=F32,
                               precision=lax.Precision.HIGHEST) + b2_ref[...]
        h2 = (mm + dvec).reshape(K, BLK, D)
        node = functools.reduce(jnp.maximum, [h2[k] for k in range(K)])
        bat = bat_ref[...]
        rows = []
        for c in range(NB):
            w = jnp.where(bat == c, node, -_INF)
            rows.append(jnp.max(w.reshape(BLK // 8, 8, D), axis=0))
        pool_ref[...] = jnp.maximum(pool_ref[...], jnp.concatenate(rows, axis=0))

    @pl.when(i == 2 * NBLK - 1)
    def _():
        p8 = pool_ref[...].reshape(NB, 8, D)
        p = functools.reduce(jnp.maximum, [p8[:, k, :] for k in range(8)])
        t = lax.dot_general(p, l1w_ref[...], dn, preferred_element_type=F32,
                            precision=lax.Precision.HIGHEST) + l1b_ref[...]
        t = jnp.maximum(t, 0.0)
        o = lax.dot_general(t, l2w_ref[...], dn, preferred_element_type=F32,
                            precision=lax.Precision.HIGHEST) + l2b_ref[...]
        nrm = jnp.sqrt(jnp.sum(o * o, axis=1, keepdims=True))
        out_ref[...] = o / jnp.maximum(nrm, 1e-12)


def kernel(x, batch, W1, b1, bn_gamma, bn_beta, W2, b2, L1w, L1b, L2w, L2b):
    batch = batch.astype(jnp.int32)
    Wd = W1[:, :D] - W1[:, D:]
    Wb = W1[:, D:]

    xn, A, Bm, sq = pl.pallas_call(
        _prep_body,
        out_shape=[jax.ShapeDtypeStruct((N, D), F32)] * 3
        + [jax.ShapeDtypeStruct((N, 1), F32)],
    )(x, Wd, Wb)

    rng = jnp.arange(NB, dtype=batch.dtype)
    cs = jnp.searchsorted(batch, rng, side='left').astype(jnp.int32)
    ce = jnp.searchsorted(batch, rng, side='right').astype(jnp.int32)
    bfirst = batch[::BLK]
    blast = batch[BLK - 1::BLK]
    starts = cs[bfirst]
    ends = ce[blast]
    u = jnp.minimum(starts // _WSUB, (N - WIN) // _WSUB).astype(jnp.int32)
    fits = jnp.all(ends - u * _WSUB <= WIN) & jnp.all(ce - cs >= K)

    batch_r = batch.reshape(N, 1)
    batch_c = batch.reshape(1, N)
    sqj = sq.reshape(1, N)

    def _full_knn():
        return pl.pallas_call(
            _knn_body,
            grid=(NBLK,),
            in_specs=[
                pl.BlockSpec((BLK, D), lambda i: (i, 0)),
                pl.BlockSpec((BLK, 1), lambda i: (i, 0)),
                pl.BlockSpec((N, D), lambda i: (0, 0)),
                pl.BlockSpec((1, N), lambda i: (0, 0)),
                pl.BlockSpec((BLK, 1), lambda i: (i, 0)),
                pl.BlockSpec((1, N), lambda i: (0, 0)),
            ],
            out_specs=pl.BlockSpec((BLK, K), lambda i: (i, 0)),
            out_shape=jax.ShapeDtypeStruct((N, K), jnp.int32),
        )(xn, batch_r, xn, batch_c, sq, sqj)

    idx = lax.cond(fits,
                   lambda: _knn_windowed(u, xn, batch_r, batch_c, sq, sqj),
                   _full_knn)

    Bg = _gather_rows(Bm, idx.T.reshape(N * K))
    Bg3 = Bg.reshape(K, N, D)
    b13 = b1.reshape(1, 1, D)

    out = pl.pallas_call(
        _edge_body,
        grid=(2 * NBLK,),
        in_specs=[
            pl.BlockSpec((K, BLK, D), lambda i: (0, i % NBLK, 0)),
            pl.BlockSpec((BLK, D), lambda i: (i % NBLK, 0)),
            pl.BlockSpec((1, 1, D), lambda i: (0, 0, 0)),
            pl.BlockSpec((1, D), lambda i: (0, 0)),
            pl.BlockSpec((1, D), lambda i: (0, 0)),
            pl.BlockSpec((D, D), lambda i: (0, 0)),
            pl.BlockSpec((1, D), lambda i: (0, 0)),
            pl.BlockSpec((BLK, 1), lambda i: (i % NBLK, 0)),
            pl.BlockSpec((D, D), lambda i: (0, 0)),
            pl.BlockSpec((1, D), lambda i: (0, 0)),
            pl.BlockSpec((D, D), lambda i: (0, 0)),
            pl.BlockSpec((1, D), lambda i: (0, 0)),
        ],
        out_specs=pl.BlockSpec((NB, D), lambda i: (0, 0)),
        out_shape=jax.ShapeDtypeStruct((NB, D), F32),
        scratch_shapes=[pltpu.VMEM((8, D), F32), pltpu.VMEM((8, D), F32),
                        pltpu.VMEM((NB * 8, D), F32)],
    )(Bg3, A, b13, bn_gamma.reshape(1, D), bn_beta.reshape(1, D),
      W2, b2.reshape(1, D), batch.reshape(N, 1),
      L1w, L1b.reshape(1, D), L2w, L2b.reshape(1, D))

    return out
